# baseline (speedup 1.0000x reference)
import functools

import jax
import jax.numpy as jnp
from jax import lax
from jax.experimental import pallas as pl
from jax.experimental.pallas import tpu as pltpu
from jax.experimental.pallas import tpu_sc as plsc

F32 = jnp.float32
EPS = 1e-5

SC_CORES = 2
SC_SUBCORES = 16
NW = SC_CORES * SC_SUBCORES
CH = 128


def _ln(xv, g, b):
    mu = jnp.mean(xv, axis=-1, keepdims=True)
    var = jnp.mean((xv - mu) ** 2, axis=-1, keepdims=True)
    return (xv - mu) * lax.rsqrt(var + EPS) * g + b


def _mm(a, b):
    return jnp.dot(a, b, preferred_element_type=F32)


def _mmb(a, b):
    return jnp.dot(a.astype(jnp.bfloat16), b.astype(jnp.bfloat16),
                   preferred_element_type=F32)


def _encoder_body(x, posp, pW1, pb1, pW2, pb2, pg, pbe,
                  nW1a, nW1b, nb1, nW2, nb2, ng, nbe, Wi, Wj,
                  h_out, a_out, b_out):
    p1 = _mm(posp[...], pW1[...]) + pb1[...]
    p2 = _mm(p1, pW2[...]) + pb2[...]
    msk = (lax.broadcasted_iota(jnp.int32, (1, p2.shape[-1]), 1) < 3).astype(F32)
    m3 = jnp.sum(p2 * msk, axis=-1, keepdims=True) / 3.0
    v3 = jnp.sum(((p2 - m3) * msk) ** 2, axis=-1, keepdims=True) / 3.0
    pe = ((p2 - m3) * lax.rsqrt(v3 + EPS) * pg[...] + pbe[...]) * msk
    h1 = _mm(x[...], nW1a[...]) + _mm(pe, nW1b[...]) + nb1[...]
    h2 = jnp.maximum(_mm(h1, nW2[...]) + nb2[...], 0.0)
    hh = _ln(h2, ng[...], nbe[...])
    h_out[...] = hh
    a_out[...] = _mm(hh, Wi[...])
    b_out[...] = _mm(hh, Wj[...])


def _edge_encoder_body(eattr, W1, b1, W2, b2, g, be, out):
    a1 = _mm(eattr[...], W1[...]) + b1[...]
    a2 = jnp.maximum(_mmb(a1, W2[...]) + b2[...], 0.0)
    out[...] = _ln(jnp.tanh(a2), g[...], be[...])


def _pack64(x):
    u = lax.bitcast_convert_type(x, jnp.int32)
    r = jnp.right_shift(
        u + jnp.int32(0x7FFF) + jnp.bitwise_and(jnp.right_shift(u, 16), 1),
        16)
    lo = jnp.bitwise_and(r[:, :64], jnp.int32(0xFFFF))
    hi = jnp.left_shift(r[:, 64:], 16)
    return jnp.bitwise_or(lo, hi)


def _unpack64(xp):
    lo = lax.bitcast_convert_type(jnp.left_shift(xp, 16), F32)
    hi = lax.bitcast_convert_type(jnp.bitwise_and(xp, jnp.int32(-65536)),
                                  F32)
    return lo, hi


def _proj_body(h, Wi, Wj, a_out, b_out):
    a_out[...] = _mm(h[...], Wi[...])
    b_out[...] = _mm(h[...], Wj[...])


def _edge_layer_body(gi, gj, ea, We, b1, W2, b2, g, be, out):
    t = _mmb(ea[...], We[...])
    m1 = jnp.maximum(gi[...] + gj[...] + t + b1[...], 0.0)
    m = _mmb(m1, W2[...]) + b2[...]
    out[...] = _ln(m, g[...], be[...]) + ea[...]


def _node_layer_body(h, p0, p1, p2, p3, W1a, W1b, b1, W2, b2, g, be, out):
    agg = (p0[...] + p1[...]) + (p2[...] + p3[...])
    u1 = jnp.maximum(_mm(h[...], W1a[...]) + _mm(agg, W1b[...]) + b1[...], 0.0)
    u2 = jnp.tanh(_mm(u1, W2[...]) + b2[...])
    out[...] = h[...] + _ln(u2, g[...], be[...])


def _node_proj_body(h, p0, p1, p2, p3, W1a, W1b, b1, W2, b2, g, be, Wi, Wj,
                    out, a_out, b_out):
    agg = (p0[...] + p1[...]) + (p2[...] + p3[...])
    u1 = jnp.maximum(_mm(h[...], W1a[...]) + _mm(agg, W1b[...]) + b1[...], 0.0)
    u2 = jnp.tanh(_mm(u1, W2[...]) + b2[...])
    hn = h[...] + _ln(u2, g[...], be[...])
    out[...] = hn
    a_out[...] = _mm(hn, Wi[...])
    b_out[...] = _mm(hn, Wj[...])


def _dec_body(h, W1, b1, W2, b2, W3, b3, out):
    u1 = jnp.maximum(_mm(h[...], W1[...]) + b1[...], 0.0)
    u2 = _mm(u1, W2[...]) + b2[...]
    out[...] = _mm(u2, W3[...]) + b3[...]


def _row_call(body, n_rows, block, n_in_row, mats, out_shapes):
    grid = n_rows // block

    def rb(arr):
        return pl.BlockSpec((block, arr.shape[-1]), lambda i: (i, 0))

    def whole(arr):
        return pl.BlockSpec(arr.shape, lambda i: tuple(0 for _ in arr.shape))

    def make(*args):
        in_specs = [rb(a) for a in args[:n_in_row]] + [whole(a) for a in mats]
        out_specs = [pl.BlockSpec((block, s[0][-1]), lambda i: (i, 0))
                     for s in out_shapes]
        out_shape = [jax.ShapeDtypeStruct(s, dt) for s, dt in out_shapes]
        single = len(out_shapes) == 1
        return pl.pallas_call(
            body,
            grid=(grid,),
            in_specs=in_specs,
            out_specs=out_specs[0] if single else out_specs,
            out_shape=out_shape[0] if single else out_shape,
        )(*args, *mats)

    return make


def _sc_mesh():
    return plsc.VectorSubcoreMesh(
        core_axis_name="c", subcore_axis_name="s",
        num_cores=SC_CORES, num_subcores=SC_SUBCORES)


SLOTS = 3


def _gather_pair(A, B, dst3, src3):
    _, K, _ = dst3.shape
    H = A.shape[-1]
    dt = A.dtype
    Ep = NW * K * CH
    S = SLOTS

    @functools.partial(
        pl.kernel,
        out_type=(jax.ShapeDtypeStruct((Ep, H), dt),
                  jax.ShapeDtypeStruct((Ep, H), dt)),
        mesh=_sc_mesh(),
        scratch_types=[pltpu.VMEM((K, CH), jnp.int32)] * 2
        + [pltpu.VMEM((CH, H), dt)] * (2 * S)
        + [pltpu.SemaphoreType.DMA] * (4 * S),
    )
    def k(a_hbm, b_hbm, d_hbm, s_hbm, gi_hbm, gj_hbm, dblk, sblk, *rest):
        ra, rb = rest[0:S], rest[S:2 * S]
        ga, gb = rest[2 * S:3 * S], rest[3 * S:4 * S]
        wa, wb = rest[4 * S:5 * S], rest[5 * S:6 * S]
        wid = lax.axis_index("s") * SC_CORES + lax.axis_index("c")
        pltpu.sync_copy(d_hbm.at[wid], dblk)
        pltpu.sync_copy(s_hbm.at[wid], sblk)

        def issue_gather(j, s):
            pltpu.async_copy(a_hbm.at[dblk.at[j]], ra[s], ga[s])
            pltpu.async_copy(b_hbm.at[sblk.at[j]], rb[s], gb[s])

        def wait_gather(j, s):
            pltpu.make_async_copy(a_hbm.at[dblk.at[j]], ra[s], ga[s]).wait()
            pltpu.make_async_copy(b_hbm.at[sblk.at[j]], rb[s], gb[s]).wait()

        def issue_wb(j, s):
            base = (wid * K + j) * CH
            pltpu.async_copy(ra[s], gi_hbm.at[pl.ds(base, CH)], wa[s])
            pltpu.async_copy(rb[s], gj_hbm.at[pl.ds(base, CH)], wb[s])

        def wait_wb(j, s):
            base = (wid * K + j) * CH
            pltpu.make_async_copy(ra[s], gi_hbm.at[pl.ds(base, CH)],
                                  wa[s]).wait()
            pltpu.make_async_copy(rb[s], gj_hbm.at[pl.ds(base, CH)],
                                  wb[s]).wait()

        for j in range(K):
            s = j % S
            if j >= S:
                wait_wb(j - S, s)
            issue_gather(j, s)
            if j >= S - 1:
                jj = j - (S - 1)
                wait_gather(jj, jj % S)
                issue_wb(jj, jj % S)
        for jj in range(max(0, K - (S - 1)), K):
            wait_gather(jj, jj % S)
            issue_wb(jj, jj % S)
        for jj in range(max(0, K - S), K):
            wait_wb(jj, jj % S)

    return k(A, B, dst3, src3)


def _segment_sum_2part(ea, src3, zeros, Np):
    _, K, _ = src3.shape
    H = ea.shape[-1]
    S = 2
    rpt = Np // SC_SUBCORES

    @functools.partial(
        pl.kernel,
        out_type=jax.ShapeDtypeStruct((2, Np, H), F32),
        mesh=_sc_mesh(),
        scratch_types=[pltpu.VMEM_SHARED((Np, H), F32)]
        + [pltpu.VMEM((K, CH), jnp.int32)]
        + [pltpu.VMEM((CH, H), F32)] * S
        + [pltpu.SemaphoreType.DMA] * (2 * S),
    )
    def k(ea_hbm, s_hbm, z_hbm, out_hbm, table, sblk, *rest):
        rv = rest[0:S]
        gl = rest[S:2 * S]
        sc = rest[2 * S:3 * S]
        cid = lax.axis_index("c")
        sid = lax.axis_index("s")
        wid = sid * SC_CORES + cid
        pltpu.sync_copy(z_hbm.at[pl.ds(sid * rpt, rpt)],
                        table.at[pl.ds(sid * rpt, rpt)])
        pltpu.sync_copy(s_hbm.at[wid], sblk)
        plsc.subcore_barrier()

        def issue_load(j, s):
            base = (wid * K + j) * CH
            pltpu.async_copy(ea_hbm.at[pl.ds(base, CH)], rv[s], gl[s])

        def wait_load(j, s):
            base = (wid * K + j) * CH
            pltpu.make_async_copy(ea_hbm.at[pl.ds(base, CH)], rv[s],
                                  gl[s]).wait()

        def issue_scat(j, s):
            pltpu.async_copy(rv[s], table.at[sblk.at[j]], sc[s], add=True)

        def wait_scat(j, s):
            pltpu.make_async_copy(rv[s], table.at[sblk.at[j]],
                                  sc[s]).wait()

        for j in range(K):
            s = j % S
            if j >= S:
                wait_scat(j - S, s)
            issue_load(j, s)
            if j >= S - 1:
                jj = j - (S - 1)
                wait_load(jj, jj % S)
                issue_scat(jj, jj % S)
        for jj in range(max(0, K - (S - 1)), K):
            wait_load(jj, jj % S)
            issue_scat(jj, jj % S)
        for jj in range(max(0, K - S), K):
            wait_scat(jj, jj % S)

        plsc.subcore_barrier()
        pltpu.sync_copy(table.at[pl.ds(sid * rpt, rpt)],
                        out_hbm.at[cid, pl.ds(sid * rpt, rpt)])

    return k(ea, src3, zeros)


def kernel(x, pos, edge_attr, params, edge_index):
    N, DN = x.shape
    E, DE = edge_attr.shape
    DG = pos.shape[-1]
    H = params['node']['W2'].shape[-1]
    HE = params['edge']['W2'].shape[-1]
    OUT = params['dec']['W3'].shape[-1]
    BN = 2000
    BE = 4000

    def row(v):
        return v.reshape(1, -1)

    p = params['pos']
    posp = jnp.pad(pos, ((0, 0), (0, H - DG)))
    pW1 = jnp.pad(p['W1'], ((0, H - DG), (0, 0)))
    pW2 = jnp.pad(p['W2'], ((0, 0), (0, H - DG)))
    pb2 = jnp.pad(row(p['b2']), ((0, 0), (0, H - DG)))
    pg = jnp.pad(row(p['g']), ((0, 0), (0, H - DG)))
    pbe = jnp.pad(row(p['be']), ((0, 0), (0, H - DG)))
    n = params['node']
    nW1a = n['W1'][:DN]
    nW1b = jnp.pad(n['W1'][DN:], ((0, H - DG), (0, 0)))
    lys = params['layers']
    Wis = [lp['eW1'][:H] for lp in lys]
    Wjs = [lp['eW1'][H:2 * H] for lp in lys]
    h, A, B = _row_call(
        _encoder_body, N, BN, 2,
        [pW1, row(p['b1']), pW2, pb2, pg, pbe,
         nW1a, nW1b, row(n['b1']), n['W2'], row(n['b2']),
         row(n['g']), row(n['be']), Wis[0], Wjs[0]],
        [((N, H), F32), ((N, HE), F32), ((N, HE), F32)])(x, posp)

    E2 = E // 2
    n_chunks = E2 // CH
    k_per = (n_chunks + NW - 1) // NW
    pad_e = NW * k_per * CH - E2
    Ep = E2 + pad_e
    Npq = 8 * SC_SUBCORES
    Np = Npq * ((N + Npq - 1) // Npq)
    zeros = jnp.zeros((Np, HE), F32)
    e = params['edge']
    eas = [_row_call(
        _edge_encoder_body, E2, BE, 1,
        [e['W1'], row(e['b1']), e['W2'], row(e['b2']), row(e['g']),
         row(e['be'])],
        [((Ep, HE), F32)])(eattr_h)
        for eattr_h in (edge_attr[:E2], edge_attr[E2:])]

    src = edge_index[0].astype(jnp.int32)
    dst = edge_index[1].astype(jnp.int32)
    idx3 = [jnp.pad(v, (0, pad_e)).reshape(NW, k_per, CH)
            for v in (dst[:E2], dst[E2:])]
    idx3 += [jnp.pad(v, (0, pad_e), constant_values=Np - 1)
             .reshape(NW, k_per, CH) for v in (src[:E2], src[E2:])]

    for li, lp in enumerate(lys):
        We = lp['eW1'][2 * H:]
        edge_call = _row_call(
            _edge_layer_body, E2, BE, 3,
            [We, row(lp['eb1']), lp['eW2'], row(lp['eb2']), row(lp['eg']),
             row(lp['ebe'])],
            [((Ep, HE), F32)])
        gath = [_gather_pair(A, B, idx3[hf], idx3[2 + hf])
                for hf in (0, 1)]
        eas = [edge_call(gath[hf][0], gath[hf][1], eas[hf]) for hf in (0, 1)]
        parts = [_segment_sum_2part(eas[hf], idx3[2 + hf], zeros, Np)
                 for hf in (0, 1)]
        pv = (parts[0][0, :N], parts[0][1, :N], parts[1][0, :N],
              parts[1][1, :N])
        if li + 1 < len(lys):
            h, A, B = _row_call(
                _node_proj_body, N, BN, 5,
                [lp['nW1'][:H], lp['nW1'][H:], row(lp['nb1']), lp['nW2'],
                 row(lp['nb2']), row(lp['ng']), row(lp['nbe']),
                 Wis[li + 1], Wjs[li + 1]],
                [((N, H), F32), ((N, HE), F32), ((N, HE), F32)])(h, *pv)
        else:
            h = _row_call(
                _node_layer_body, N, BN, 5,
                [lp['nW1'][:H], lp['nW1'][H:], row(lp['nb1']), lp['nW2'],
                 row(lp['nb2']), row(lp['ng']), row(lp['nbe'])],
                [((N, H), F32)])(h, *pv)

    d = params['dec']
    dW3 = jnp.pad(d['W3'], ((0, 0), (0, H - OUT)))
    db3 = jnp.pad(row(d['b3']), ((0, 0), (0, H - OUT)))
    outp = _row_call(
        _dec_body, N, BN, 1,
        [d['W1'], row(d['b1']), d['W2'], row(d['b2']), dW3, db3],
        [((N, H), F32)])(h)
    return outp[:, :OUT]


# speedup vs baseline: 1.5892x; 1.5892x over previous
import functools

import jax
import jax.numpy as jnp
from jax import lax
from jax.experimental import pallas as pl
from jax.experimental.pallas import tpu as pltpu
from jax.experimental.pallas import tpu_sc as plsc

F32 = jnp.float32
EPS = 1e-5

SC_CORES = 2
SC_SUBCORES = 16
NW = SC_CORES * SC_SUBCORES
CH = 128


def _ln(xv, g, b):
    mu = jnp.mean(xv, axis=-1, keepdims=True)
    var = jnp.mean((xv - mu) ** 2, axis=-1, keepdims=True)
    return (xv - mu) * lax.rsqrt(var + EPS) * g + b


def _mm(a, b):
    return jnp.dot(a, b, preferred_element_type=F32)


def _mmb(a, b):
    return jnp.dot(a.astype(jnp.bfloat16), b.astype(jnp.bfloat16),
                   preferred_element_type=F32)


def _encoder_body(x, posp, pW1, pb1, pW2, pb2, pg, pbe,
                  nW1a, nW1b, nb1, nW2, nb2, ng, nbe, Wi, Wj,
                  h_out, a_out, b_out):
    p1 = _mm(posp[...], pW1[...]) + pb1[...]
    p2 = _mm(p1, pW2[...]) + pb2[...]
    msk = (lax.broadcasted_iota(jnp.int32, (1, p2.shape[-1]), 1) < 3).astype(F32)
    m3 = jnp.sum(p2 * msk, axis=-1, keepdims=True) / 3.0
    v3 = jnp.sum(((p2 - m3) * msk) ** 2, axis=-1, keepdims=True) / 3.0
    pe = ((p2 - m3) * lax.rsqrt(v3 + EPS) * pg[...] + pbe[...]) * msk
    h1 = _mm(x[...], nW1a[...]) + _mm(pe, nW1b[...]) + nb1[...]
    h2 = jnp.maximum(_mm(h1, nW2[...]) + nb2[...], 0.0)
    hh = _ln(h2, ng[...], nbe[...])
    h_out[...] = hh
    a_out[...] = _mm(hh, Wi[...])
    b_out[...] = _mm(hh, Wj[...])


def _edge_encoder_body(eattr, W1, b1, W2, b2, g, be, out):
    a1 = _mm(eattr[...], W1[...]) + b1[...]
    a2 = jnp.maximum(_mmb(a1, W2[...]) + b2[...], 0.0)
    out[...] = _ln(jnp.tanh(a2), g[...], be[...])


def _pack64(x):
    u = lax.bitcast_convert_type(x, jnp.int32)
    r = jnp.right_shift(
        u + jnp.int32(0x7FFF) + jnp.bitwise_and(jnp.right_shift(u, 16), 1),
        16)
    lo = jnp.bitwise_and(r[:, :64], jnp.int32(0xFFFF))
    hi = jnp.left_shift(r[:, 64:], 16)
    return jnp.bitwise_or(lo, hi)


def _unpack64(xp):
    lo = lax.bitcast_convert_type(jnp.left_shift(xp, 16), F32)
    hi = lax.bitcast_convert_type(jnp.bitwise_and(xp, jnp.int32(-65536)),
                                  F32)
    return lo, hi


def _proj_body(h, Wi, Wj, a_out, b_out):
    a_out[...] = _mm(h[...], Wi[...])
    b_out[...] = _mm(h[...], Wj[...])


def _edge_layer_body(gi, gj, ea, We, b1, W2, b2, g, be, out):
    t = _mmb(ea[...], We[...])
    m1 = jnp.maximum(gi[...] + gj[...] + t + b1[...], 0.0)
    m = _mmb(m1, W2[...]) + b2[...]
    out[...] = _ln(m, g[...], be[...]) + ea[...]


def _node_layer_body(h, p0, p1, p2, p3, W1a, W1b, b1, W2, b2, g, be, out):
    agg = (p0[...] + p1[...]) + (p2[...] + p3[...])
    u1 = jnp.maximum(_mm(h[...], W1a[...]) + _mm(agg, W1b[...]) + b1[...], 0.0)
    u2 = jnp.tanh(_mm(u1, W2[...]) + b2[...])
    out[...] = h[...] + _ln(u2, g[...], be[...])


def _node_proj_body(h, p0, p1, p2, p3, W1a, W1b, b1, W2, b2, g, be, Wi, Wj,
                    out, a_out, b_out):
    agg = (p0[...] + p1[...]) + (p2[...] + p3[...])
    u1 = jnp.maximum(_mm(h[...], W1a[...]) + _mm(agg, W1b[...]) + b1[...], 0.0)
    u2 = jnp.tanh(_mm(u1, W2[...]) + b2[...])
    hn = h[...] + _ln(u2, g[...], be[...])
    out[...] = hn
    a_out[...] = _mm(hn, Wi[...])
    b_out[...] = _mm(hn, Wj[...])


def _dec_body(h, W1, b1, W2, b2, W3, b3, out):
    u1 = jnp.maximum(_mm(h[...], W1[...]) + b1[...], 0.0)
    u2 = _mm(u1, W2[...]) + b2[...]
    out[...] = _mm(u2, W3[...]) + b3[...]


def _row_call(body, n_rows, block, n_in_row, mats, out_shapes):
    grid = n_rows // block

    def rb(arr):
        return pl.BlockSpec((block, arr.shape[-1]), lambda i: (i, 0))

    def whole(arr):
        return pl.BlockSpec(arr.shape, lambda i: tuple(0 for _ in arr.shape))

    def make(*args):
        in_specs = [rb(a) for a in args[:n_in_row]] + [whole(a) for a in mats]
        out_specs = [pl.BlockSpec((block, s[0][-1]), lambda i: (i, 0))
                     for s in out_shapes]
        out_shape = [jax.ShapeDtypeStruct(s, dt) for s, dt in out_shapes]
        single = len(out_shapes) == 1
        return pl.pallas_call(
            body,
            grid=(grid,),
            in_specs=in_specs,
            out_specs=out_specs[0] if single else out_specs,
            out_shape=out_shape[0] if single else out_shape,
        )(*args, *mats)

    return make


def _sc_mesh():
    return plsc.VectorSubcoreMesh(
        core_axis_name="c", subcore_axis_name="s",
        num_cores=SC_CORES, num_subcores=SC_SUBCORES)


def _gather_pair(A, B, dst3, src3, E):
    n_chunks = E // CH
    k_per = (n_chunks + NW - 1) // NW
    H = A.shape[-1]
    dt = A.dtype

    @functools.partial(
        pl.kernel,
        out_type=(jax.ShapeDtypeStruct((E, H), dt),
                  jax.ShapeDtypeStruct((E, H), dt)),
        mesh=_sc_mesh(),
        scratch_types=[
            pltpu.VMEM((k_per, CH), jnp.int32),
            pltpu.VMEM((k_per, CH), jnp.int32),
            pltpu.VMEM((CH, H), dt),
            pltpu.VMEM((CH, H), dt),
            pltpu.VMEM((CH, H), dt),
            pltpu.VMEM((CH, H), dt),
        ] + [pltpu.SemaphoreType.DMA] * 8,
    )
    def k(a_hbm, b_hbm, d_hbm, s_hbm, gi_hbm, gj_hbm,
          dblk, sblk, ra0, ra1, rb0, rb1,
          ga0, ga1, gb0, gb1, wa0, wa1, wb0, wb1):
        wid = lax.axis_index("s") * SC_CORES + lax.axis_index("c")
        pltpu.sync_copy(d_hbm.at[wid], dblk)
        pltpu.sync_copy(s_hbm.at[wid], sblk)
        nv = jnp.maximum(0, jnp.minimum(k_per, n_chunks - wid * k_per))
        ra, rb = (ra0, ra1), (rb0, rb1)
        ga, gb = (ga0, ga1), (gb0, gb1)
        wa, wb = (wa0, wa1), (wb0, wb1)

        def on_parity(j, fn):
            @pl.when(lax.rem(j, 2) == 0)
            def _():
                fn(0)

            @pl.when(lax.rem(j, 2) == 1)
            def _():
                fn(1)

        def issue_gather(j, s):
            pltpu.async_copy(a_hbm.at[dblk.at[j]], ra[s], ga[s])
            pltpu.async_copy(b_hbm.at[sblk.at[j]], rb[s], gb[s])

        def wait_gather(j, s):
            pltpu.make_async_copy(a_hbm.at[dblk.at[j]], ra[s], ga[s]).wait()
            pltpu.make_async_copy(b_hbm.at[sblk.at[j]], rb[s], gb[s]).wait()

        def issue_wb(j, s):
            base = (wid * k_per + j) * CH
            pltpu.async_copy(ra[s], gi_hbm.at[pl.ds(base, CH)], wa[s])
            pltpu.async_copy(rb[s], gj_hbm.at[pl.ds(base, CH)], wb[s])

        def wait_wb(j, s):
            base = (wid * k_per + j) * CH
            pltpu.make_async_copy(ra[s], gi_hbm.at[pl.ds(base, CH)],
                                  wa[s]).wait()
            pltpu.make_async_copy(rb[s], gj_hbm.at[pl.ds(base, CH)],
                                  wb[s]).wait()

        @pl.when(nv > 0)
        def _():
            issue_gather(0, 0)

        def body(j, carry):
            @pl.when(j + 1 < nv)
            def _():
                def ahead(ns):
                    @pl.when(j >= 1)
                    def _():
                        wait_wb(j - 1, ns)

                    issue_gather(j + 1, ns)

                on_parity(j + 1, ahead)

            def cur(s):
                wait_gather(j, s)
                issue_wb(j, s)

            on_parity(j, cur)
            return carry

        lax.fori_loop(0, nv, body, 0)

        @pl.when(nv >= 2)
        def _():
            on_parity(nv - 2, lambda s: wait_wb(nv - 2, s))

        @pl.when(nv >= 1)
        def _():
            on_parity(nv - 1, lambda s: wait_wb(nv - 1, s))

    return k(A, B, dst3, src3)


def _segment_sum_2part(ea, src3, zeros, Np, E):
    n_chunks = E // CH
    k_per = (n_chunks + NW - 1) // NW
    H = ea.shape[-1]
    rpt = Np // SC_SUBCORES

    @functools.partial(
        pl.kernel,
        out_type=jax.ShapeDtypeStruct((2, Np, H), F32),
        mesh=_sc_mesh(),
        scratch_types=[
            pltpu.VMEM((k_per, CH), jnp.int32),
            pltpu.VMEM((CH, H), F32),
            pltpu.VMEM((CH, H), F32),
            pltpu.VMEM_SHARED((Np, H), F32),
        ] + [pltpu.SemaphoreType.DMA] * 4,
    )
    def k(ea_hbm, s_hbm, z_hbm, out_hbm, sblk, rv0, rv1, table,
          gl0, gl1, sc0, sc1):
        cid = lax.axis_index("c")
        sid = lax.axis_index("s")
        wid = sid * SC_CORES + cid
        pltpu.sync_copy(z_hbm.at[pl.ds(sid * rpt, rpt)],
                        table.at[pl.ds(sid * rpt, rpt)])
        pltpu.sync_copy(s_hbm.at[wid], sblk)
        plsc.subcore_barrier()
        nv = jnp.maximum(0, jnp.minimum(k_per, n_chunks - wid * k_per))
        rv = (rv0, rv1)
        gl = (gl0, gl1)
        sc = (sc0, sc1)

        def on_parity(j, fn):
            @pl.when(lax.rem(j, 2) == 0)
            def _():
                fn(0)

            @pl.when(lax.rem(j, 2) == 1)
            def _():
                fn(1)

        def issue_load(j, s):
            base = (wid * k_per + j) * CH
            pltpu.async_copy(ea_hbm.at[pl.ds(base, CH)], rv[s], gl[s])

        def wait_load(j, s):
            base = (wid * k_per + j) * CH
            pltpu.make_async_copy(ea_hbm.at[pl.ds(base, CH)], rv[s],
                                  gl[s]).wait()

        def issue_scat(j, s):
            pltpu.async_copy(rv[s], table.at[sblk.at[j]], sc[s], add=True)

        def wait_scat(j, s):
            pltpu.make_async_copy(rv[s], table.at[sblk.at[j]],
                                  sc[s]).wait()

        @pl.when(nv > 0)
        def _():
            issue_load(0, 0)

        def body(j, carry):
            @pl.when(j + 1 < nv)
            def _():
                def ahead(ns):
                    @pl.when(j >= 1)
                    def _():
                        wait_scat(j - 1, ns)

                    issue_load(j + 1, ns)

                on_parity(j + 1, ahead)

            def cur(s):
                wait_load(j, s)
                issue_scat(j, s)

            on_parity(j, cur)
            return carry

        lax.fori_loop(0, nv, body, 0)

        @pl.when(nv >= 2)
        def _():
            on_parity(nv - 2, lambda s: wait_scat(nv - 2, s))

        @pl.when(nv >= 1)
        def _():
            on_parity(nv - 1, lambda s: wait_scat(nv - 1, s))

        plsc.subcore_barrier()
        pltpu.sync_copy(table.at[pl.ds(sid * rpt, rpt)],
                        out_hbm.at[cid, pl.ds(sid * rpt, rpt)])

    return k(ea, src3, zeros)


def kernel(x, pos, edge_attr, params, edge_index):
    N, DN = x.shape
    E, DE = edge_attr.shape
    DG = pos.shape[-1]
    H = params['node']['W2'].shape[-1]
    HE = params['edge']['W2'].shape[-1]
    OUT = params['dec']['W3'].shape[-1]
    BN = 2000
    BE = 8000

    def row(v):
        return v.reshape(1, -1)

    p = params['pos']
    posp = jnp.pad(pos, ((0, 0), (0, H - DG)))
    pW1 = jnp.pad(p['W1'], ((0, H - DG), (0, 0)))
    pW2 = jnp.pad(p['W2'], ((0, 0), (0, H - DG)))
    pb2 = jnp.pad(row(p['b2']), ((0, 0), (0, H - DG)))
    pg = jnp.pad(row(p['g']), ((0, 0), (0, H - DG)))
    pbe = jnp.pad(row(p['be']), ((0, 0), (0, H - DG)))
    n = params['node']
    nW1a = n['W1'][:DN]
    nW1b = jnp.pad(n['W1'][DN:], ((0, H - DG), (0, 0)))
    lys = params['layers']
    Wis = [lp['eW1'][:H] for lp in lys]
    Wjs = [lp['eW1'][H:2 * H] for lp in lys]
    h, A, B = _row_call(
        _encoder_body, N, BN, 2,
        [pW1, row(p['b1']), pW2, pb2, pg, pbe,
         nW1a, nW1b, row(n['b1']), n['W2'], row(n['b2']),
         row(n['g']), row(n['be']), Wis[0], Wjs[0]],
        [((N, H), F32), ((N, HE), F32), ((N, HE), F32)])(x, posp)

    E2 = E // 2
    e = params['edge']
    eas = [_row_call(
        _edge_encoder_body, E2, BE, 1,
        [e['W1'], row(e['b1']), e['W2'], row(e['b2']), row(e['g']),
         row(e['be'])],
        [((E2, HE), F32)])(eattr_h)
        for eattr_h in (edge_attr[:E2], edge_attr[E2:])]

    n_chunks = E2 // CH
    k_per = (n_chunks + NW - 1) // NW
    pad_e = NW * k_per * CH - E2
    src = edge_index[0].astype(jnp.int32)
    dst = edge_index[1].astype(jnp.int32)
    idx3 = [jnp.pad(v, (0, pad_e)).reshape(NW, k_per, CH)
            for v in (dst[:E2], dst[E2:], src[:E2], src[E2:])]
    Np = 8 * SC_SUBCORES * ((N + 8 * SC_SUBCORES - 1) // (8 * SC_SUBCORES))
    zeros = jnp.zeros((Np, HE), F32)

    for li, lp in enumerate(lys):
        We = lp['eW1'][2 * H:]
        edge_call = _row_call(
            _edge_layer_body, E2, BE, 3,
            [We, row(lp['eb1']), lp['eW2'], row(lp['eb2']), row(lp['eg']),
             row(lp['ebe'])],
            [((E2, HE), F32)])
        gath = [_gather_pair(A, B, idx3[hf], idx3[2 + hf], E2)
                for hf in (0, 1)]
        eas = [edge_call(gath[hf][0], gath[hf][1], eas[hf]) for hf in (0, 1)]
        parts = [_segment_sum_2part(eas[hf], idx3[2 + hf], zeros, Np, E2)
                 for hf in (0, 1)]
        pv = (parts[0][0, :N], parts[0][1, :N], parts[1][0, :N],
              parts[1][1, :N])
        if li + 1 < len(lys):
            h, A, B = _row_call(
                _node_proj_body, N, BN, 5,
                [lp['nW1'][:H], lp['nW1'][H:], row(lp['nb1']), lp['nW2'],
                 row(lp['nb2']), row(lp['ng']), row(lp['nbe']),
                 Wis[li + 1], Wjs[li + 1]],
                [((N, H), F32), ((N, HE), F32), ((N, HE), F32)])(h, *pv)
        else:
            h = _row_call(
                _node_layer_body, N, BN, 5,
                [lp['nW1'][:H], lp['nW1'][H:], row(lp['nb1']), lp['nW2'],
                 row(lp['nb2']), row(lp['ng']), row(lp['nbe'])],
                [((N, H), F32)])(h, *pv)

    d = params['dec']
    dW3 = jnp.pad(d['W3'], ((0, 0), (0, H - OUT)))
    db3 = jnp.pad(row(d['b3']), ((0, 0), (0, H - OUT)))
    outp = _row_call(
        _dec_body, N, BN, 1,
        [d['W1'], row(d['b1']), d['W2'], row(d['b2']), dW3, db3],
        [((N, H), F32)])(h)
    return outp[:, :OUT]


# speedup vs baseline: 1.6106x; 1.0134x over previous
import functools

import jax
import jax.numpy as jnp
from jax import lax
from jax.experimental import pallas as pl
from jax.experimental.pallas import tpu as pltpu
from jax.experimental.pallas import tpu_sc as plsc

F32 = jnp.float32
EPS = 1e-5

SC_CORES = 2
SC_SUBCORES = 16
NW = SC_CORES * SC_SUBCORES
CH = 128


def _ln(xv, g, b):
    mu = jnp.mean(xv, axis=-1, keepdims=True)
    var = jnp.mean((xv - mu) ** 2, axis=-1, keepdims=True)
    return (xv - mu) * lax.rsqrt(var + EPS) * g + b


def _mm(a, b):
    return jnp.dot(a, b, preferred_element_type=F32)


def _mmb(a, b):
    return jnp.dot(a.astype(jnp.bfloat16), b.astype(jnp.bfloat16),
                   preferred_element_type=F32)


def _encoder_body(x, posp, pW1, pb1, pW2, pb2, pg, pbe,
                  nW1a, nW1b, nb1, nW2, nb2, ng, nbe, Wi, Wj,
                  h_out, a_out, b_out):
    p1 = _mm(posp[...], pW1[...]) + pb1[...]
    p2 = _mm(p1, pW2[...]) + pb2[...]
    msk = (lax.broadcasted_iota(jnp.int32, (1, p2.shape[-1]), 1) < 3).astype(F32)
    m3 = jnp.sum(p2 * msk, axis=-1, keepdims=True) / 3.0
    v3 = jnp.sum(((p2 - m3) * msk) ** 2, axis=-1, keepdims=True) / 3.0
    pe = ((p2 - m3) * lax.rsqrt(v3 + EPS) * pg[...] + pbe[...]) * msk
    h1 = _mm(x[...], nW1a[...]) + _mm(pe, nW1b[...]) + nb1[...]
    h2 = jnp.maximum(_mm(h1, nW2[...]) + nb2[...], 0.0)
    hh = _ln(h2, ng[...], nbe[...])
    h_out[...] = hh
    a_out[...] = _mm(hh, Wi[...])
    b_out[...] = _mm(hh, Wj[...])


def _edge_encoder_body(eattr, W1, b1, W2, b2, g, be, out):
    a1 = _mm(eattr[...], W1[...]) + b1[...]
    a2 = jnp.maximum(_mmb(a1, W2[...]) + b2[...], 0.0)
    out[...] = _ln(jnp.tanh(a2), g[...], be[...])


def _pack64(x):
    u = lax.bitcast_convert_type(x, jnp.int32)
    r = jnp.right_shift(
        u + jnp.int32(0x7FFF) + jnp.bitwise_and(jnp.right_shift(u, 16), 1),
        16)
    lo = jnp.bitwise_and(r[:, :64], jnp.int32(0xFFFF))
    hi = jnp.left_shift(r[:, 64:], 16)
    return jnp.bitwise_or(lo, hi)


def _unpack64(xp):
    lo = lax.bitcast_convert_type(jnp.left_shift(xp, 16), F32)
    hi = lax.bitcast_convert_type(jnp.bitwise_and(xp, jnp.int32(-65536)),
                                  F32)
    return lo, hi


def _proj_body(h, Wi, Wj, a_out, b_out):
    a_out[...] = _mm(h[...], Wi[...])
    b_out[...] = _mm(h[...], Wj[...])


def _edge_layer_body(gi, gj, ea, We, b1, W2, b2, g, be, out):
    t = _mmb(ea[...], We[...])
    m1 = jnp.maximum(gi[...] + gj[...] + t + b1[...], 0.0)
    m = _mmb(m1, W2[...]) + b2[...]
    out[...] = _ln(m, g[...], be[...]) + ea[...]


def _node_layer_body(h, p0, p1, W1a, W1b, b1, W2, b2, g, be, out):
    agg = p0[...] + p1[...]
    u1 = jnp.maximum(_mm(h[...], W1a[...]) + _mm(agg, W1b[...]) + b1[...], 0.0)
    u2 = jnp.tanh(_mm(u1, W2[...]) + b2[...])
    out[...] = h[...] + _ln(u2, g[...], be[...])


def _node_proj_body(h, p0, p1, W1a, W1b, b1, W2, b2, g, be, Wi, Wj,
                    out, a_out, b_out):
    agg = p0[...] + p1[...]
    u1 = jnp.maximum(_mm(h[...], W1a[...]) + _mm(agg, W1b[...]) + b1[...], 0.0)
    u2 = jnp.tanh(_mm(u1, W2[...]) + b2[...])
    hn = h[...] + _ln(u2, g[...], be[...])
    out[...] = hn
    a_out[...] = _mm(hn, Wi[...])
    b_out[...] = _mm(hn, Wj[...])


def _dec_body(h, W1, b1, W2, b2, W3, b3, out):
    u1 = jnp.maximum(_mm(h[...], W1[...]) + b1[...], 0.0)
    u2 = _mm(u1, W2[...]) + b2[...]
    out[...] = _mm(u2, W3[...]) + b3[...]


def _row_call(body, n_rows, block, n_in_row, mats, out_shapes):
    grid = n_rows // block

    def rb(arr):
        return pl.BlockSpec((block, arr.shape[-1]), lambda i: (i, 0))

    def whole(arr):
        return pl.BlockSpec(arr.shape, lambda i: tuple(0 for _ in arr.shape))

    def make(*args):
        in_specs = [rb(a) for a in args[:n_in_row]] + [whole(a) for a in mats]
        out_specs = [pl.BlockSpec((block, s[0][-1]), lambda i: (i, 0))
                     for s in out_shapes]
        out_shape = [jax.ShapeDtypeStruct(s, dt) for s, dt in out_shapes]
        single = len(out_shapes) == 1
        return pl.pallas_call(
            body,
            grid=(grid,),
            in_specs=in_specs,
            out_specs=out_specs[0] if single else out_specs,
            out_shape=out_shape[0] if single else out_shape,
        )(*args, *mats)

    return make


def _sc_mesh():
    return plsc.VectorSubcoreMesh(
        core_axis_name="c", subcore_axis_name="s",
        num_cores=SC_CORES, num_subcores=SC_SUBCORES)


def _gather_pair(A, B, dst3, src3, E):
    n_chunks = E // CH
    k_per = (n_chunks + NW - 1) // NW
    H = A.shape[-1]
    dt = A.dtype

    @functools.partial(
        pl.kernel,
        out_type=(jax.ShapeDtypeStruct((E, H), dt),
                  jax.ShapeDtypeStruct((E, H), dt)),
        mesh=_sc_mesh(),
        scratch_types=[
            pltpu.VMEM((k_per, CH), jnp.int32),
            pltpu.VMEM((k_per, CH), jnp.int32),
            pltpu.VMEM((CH, H), dt),
            pltpu.VMEM((CH, H), dt),
            pltpu.VMEM((CH, H), dt),
            pltpu.VMEM((CH, H), dt),
        ] + [pltpu.SemaphoreType.DMA] * 8,
    )
    def k(a_hbm, b_hbm, d_hbm, s_hbm, gi_hbm, gj_hbm,
          dblk, sblk, ra0, ra1, rb0, rb1,
          ga0, ga1, gb0, gb1, wa0, wa1, wb0, wb1):
        wid = lax.axis_index("s") * SC_CORES + lax.axis_index("c")
        pltpu.sync_copy(d_hbm.at[wid], dblk)
        pltpu.sync_copy(s_hbm.at[wid], sblk)
        nv = jnp.maximum(0, jnp.minimum(k_per, n_chunks - wid * k_per))
        ra, rb = (ra0, ra1), (rb0, rb1)
        ga, gb = (ga0, ga1), (gb0, gb1)
        wa, wb = (wa0, wa1), (wb0, wb1)

        def on_parity(j, fn):
            @pl.when(lax.rem(j, 2) == 0)
            def _():
                fn(0)

            @pl.when(lax.rem(j, 2) == 1)
            def _():
                fn(1)

        def issue_gather(j, s):
            pltpu.async_copy(a_hbm.at[dblk.at[j]], ra[s], ga[s])
            pltpu.async_copy(b_hbm.at[sblk.at[j]], rb[s], gb[s])

        def wait_gather(j, s):
            pltpu.make_async_copy(a_hbm.at[dblk.at[j]], ra[s], ga[s]).wait()
            pltpu.make_async_copy(b_hbm.at[sblk.at[j]], rb[s], gb[s]).wait()

        def issue_wb(j, s):
            base = (wid * k_per + j) * CH
            pltpu.async_copy(ra[s], gi_hbm.at[pl.ds(base, CH)], wa[s])
            pltpu.async_copy(rb[s], gj_hbm.at[pl.ds(base, CH)], wb[s])

        def wait_wb(j, s):
            base = (wid * k_per + j) * CH
            pltpu.make_async_copy(ra[s], gi_hbm.at[pl.ds(base, CH)],
                                  wa[s]).wait()
            pltpu.make_async_copy(rb[s], gj_hbm.at[pl.ds(base, CH)],
                                  wb[s]).wait()

        @pl.when(nv > 0)
        def _():
            issue_gather(0, 0)

        def body(j, carry):
            @pl.when(j + 1 < nv)
            def _():
                def ahead(ns):
                    @pl.when(j >= 1)
                    def _():
                        wait_wb(j - 1, ns)

                    issue_gather(j + 1, ns)

                on_parity(j + 1, ahead)

            def cur(s):
                wait_gather(j, s)
                issue_wb(j, s)

            on_parity(j, cur)
            return carry

        lax.fori_loop(0, nv, body, 0)

        @pl.when(nv >= 2)
        def _():
            on_parity(nv - 2, lambda s: wait_wb(nv - 2, s))

        @pl.when(nv >= 1)
        def _():
            on_parity(nv - 1, lambda s: wait_wb(nv - 1, s))

    return k(A, B, dst3, src3)


def _segment_sum_2part(ea, src3, zeros, Np, E):
    n_chunks = E // CH
    k_per = (n_chunks + NW - 1) // NW
    H = ea.shape[-1]
    rpt = Np // SC_SUBCORES

    @functools.partial(
        pl.kernel,
        out_type=jax.ShapeDtypeStruct((2, Np, H), F32),
        mesh=_sc_mesh(),
        scratch_types=[
            pltpu.VMEM((k_per, CH), jnp.int32),
            pltpu.VMEM((CH, H), F32),
            pltpu.VMEM((CH, H), F32),
            pltpu.VMEM_SHARED((Np, H), F32),
        ] + [pltpu.SemaphoreType.DMA] * 4,
    )
    def k(ea_hbm, s_hbm, z_hbm, out_hbm, sblk, rv0, rv1, table,
          gl0, gl1, sc0, sc1):
        cid = lax.axis_index("c")
        sid = lax.axis_index("s")
        wid = sid * SC_CORES + cid
        pltpu.sync_copy(z_hbm.at[pl.ds(sid * rpt, rpt)],
                        table.at[pl.ds(sid * rpt, rpt)])
        pltpu.sync_copy(s_hbm.at[wid], sblk)
        plsc.subcore_barrier()
        nv = jnp.maximum(0, jnp.minimum(k_per, n_chunks - wid * k_per))
        rv = (rv0, rv1)
        gl = (gl0, gl1)
        sc = (sc0, sc1)

        def on_parity(j, fn):
            @pl.when(lax.rem(j, 2) == 0)
            def _():
                fn(0)

            @pl.when(lax.rem(j, 2) == 1)
            def _():
                fn(1)

        def issue_load(j, s):
            base = (wid * k_per + j) * CH
            pltpu.async_copy(ea_hbm.at[pl.ds(base, CH)], rv[s], gl[s])

        def wait_load(j, s):
            base = (wid * k_per + j) * CH
            pltpu.make_async_copy(ea_hbm.at[pl.ds(base, CH)], rv[s],
                                  gl[s]).wait()

        def issue_scat(j, s):
            pltpu.async_copy(rv[s], table.at[sblk.at[j]], sc[s], add=True)

        def wait_scat(j, s):
            pltpu.make_async_copy(rv[s], table.at[sblk.at[j]],
                                  sc[s]).wait()

        @pl.when(nv > 0)
        def _():
            issue_load(0, 0)

        def body(j, carry):
            @pl.when(j + 1 < nv)
            def _():
                def ahead(ns):
                    @pl.when(j >= 1)
                    def _():
                        wait_scat(j - 1, ns)

                    issue_load(j + 1, ns)

                on_parity(j + 1, ahead)

            def cur(s):
                wait_load(j, s)
                issue_scat(j, s)

            on_parity(j, cur)
            return carry

        lax.fori_loop(0, nv, body, 0)

        @pl.when(nv >= 2)
        def _():
            on_parity(nv - 2, lambda s: wait_scat(nv - 2, s))

        @pl.when(nv >= 1)
        def _():
            on_parity(nv - 1, lambda s: wait_scat(nv - 1, s))

        plsc.subcore_barrier()
        pltpu.sync_copy(table.at[pl.ds(sid * rpt, rpt)],
                        out_hbm.at[cid, pl.ds(sid * rpt, rpt)])

    return k(ea, src3, zeros)


def kernel(x, pos, edge_attr, params, edge_index):
    N, DN = x.shape
    E, DE = edge_attr.shape
    DG = pos.shape[-1]
    H = params['node']['W2'].shape[-1]
    HE = params['edge']['W2'].shape[-1]
    OUT = params['dec']['W3'].shape[-1]
    BN = 2000
    BE = 8000

    def row(v):
        return v.reshape(1, -1)

    p = params['pos']
    posp = jnp.pad(pos, ((0, 0), (0, H - DG)))
    pW1 = jnp.pad(p['W1'], ((0, H - DG), (0, 0)))
    pW2 = jnp.pad(p['W2'], ((0, 0), (0, H - DG)))
    pb2 = jnp.pad(row(p['b2']), ((0, 0), (0, H - DG)))
    pg = jnp.pad(row(p['g']), ((0, 0), (0, H - DG)))
    pbe = jnp.pad(row(p['be']), ((0, 0), (0, H - DG)))
    n = params['node']
    nW1a = n['W1'][:DN]
    nW1b = jnp.pad(n['W1'][DN:], ((0, H - DG), (0, 0)))
    lys = params['layers']
    Wis = [lp['eW1'][:H] for lp in lys]
    Wjs = [lp['eW1'][H:2 * H] for lp in lys]
    h, A, B = _row_call(
        _encoder_body, N, BN, 2,
        [pW1, row(p['b1']), pW2, pb2, pg, pbe,
         nW1a, nW1b, row(n['b1']), n['W2'], row(n['b2']),
         row(n['g']), row(n['be']), Wis[0], Wjs[0]],
        [((N, H), F32), ((N, HE), F32), ((N, HE), F32)])(x, posp)

    e = params['edge']
    ea = _row_call(
        _edge_encoder_body, E, BE, 1,
        [e['W1'], row(e['b1']), e['W2'], row(e['b2']), row(e['g']),
         row(e['be'])],
        [((E, HE), F32)])(edge_attr)

    n_chunks = E // CH
    k_per = (n_chunks + NW - 1) // NW
    pad_e = NW * k_per * CH - E
    src = edge_index[0].astype(jnp.int32)
    dst = edge_index[1].astype(jnp.int32)
    dst3 = jnp.pad(dst, (0, pad_e)).reshape(NW, k_per, CH)
    src3 = jnp.pad(src, (0, pad_e)).reshape(NW, k_per, CH)
    Np = 8 * SC_SUBCORES * ((N + 8 * SC_SUBCORES - 1) // (8 * SC_SUBCORES))
    zeros = jnp.zeros((Np, HE), F32)

    for li, lp in enumerate(lys):
        We = lp['eW1'][2 * H:]
        Gi, Gj = _gather_pair(A, B, dst3, src3, E)
        ea = _row_call(
            _edge_layer_body, E, BE, 3,
            [We, row(lp['eb1']), lp['eW2'], row(lp['eb2']), row(lp['eg']),
             row(lp['ebe'])],
            [((E, HE), F32)])(Gi, Gj, ea)
        parts = _segment_sum_2part(ea, src3, zeros, Np, E)
        pv = (parts[0, :N], parts[1, :N])
        if li + 1 < len(lys):
            h, A, B = _row_call(
                _node_proj_body, N, BN, 3,
                [lp['nW1'][:H], lp['nW1'][H:], row(lp['nb1']), lp['nW2'],
                 row(lp['nb2']), row(lp['ng']), row(lp['nbe']),
                 Wis[li + 1], Wjs[li + 1]],
                [((N, H), F32), ((N, HE), F32), ((N, HE), F32)])(h, *pv)
        else:
            h = _row_call(
                _node_layer_body, N, BN, 3,
                [lp['nW1'][:H], lp['nW1'][H:], row(lp['nb1']), lp['nW2'],
                 row(lp['nb2']), row(lp['ng']), row(lp['nbe'])],
                [((N, H), F32)])(h, *pv)

    d = params['dec']
    dW3 = jnp.pad(d['W3'], ((0, 0), (0, H - OUT)))
    db3 = jnp.pad(row(d['b3']), ((0, 0), (0, H - OUT)))
    outp = _row_call(
        _dec_body, N, BN, 1,
        [d['W1'], row(d['b1']), d['W2'], row(d['b2']), dW3, db3],
        [((N, H), F32)])(h)
    return outp[:, :OUT]


# speedup vs baseline: 1.6109x; 1.0002x over previous
import functools

import jax
import jax.numpy as jnp
from jax import lax
from jax.experimental import pallas as pl
from jax.experimental.pallas import tpu as pltpu
from jax.experimental.pallas import tpu_sc as plsc

F32 = jnp.float32
EPS = 1e-5

SC_CORES = 2
SC_SUBCORES = 16
NW = SC_CORES * SC_SUBCORES
CH = 128


def _ln(xv, g, b):
    mu = jnp.mean(xv, axis=-1, keepdims=True)
    var = jnp.mean((xv - mu) ** 2, axis=-1, keepdims=True)
    return (xv - mu) * lax.rsqrt(var + EPS) * g + b


def _mm(a, b):
    return jnp.dot(a, b, preferred_element_type=F32)


def _mmb(a, b):
    return jnp.dot(a.astype(jnp.bfloat16), b.astype(jnp.bfloat16),
                   preferred_element_type=F32)


def _encoder_body(x, posp, pW1, pb1, pW2, pb2, pg, pbe,
                  nW1a, nW1b, nb1, nW2, nb2, ng, nbe, Wi, Wj,
                  h_out, a_out, b_out):
    p1 = _mm(posp[...], pW1[...]) + pb1[...]
    p2 = _mm(p1, pW2[...]) + pb2[...]
    msk = (lax.broadcasted_iota(jnp.int32, (1, p2.shape[-1]), 1) < 3).astype(F32)
    m3 = jnp.sum(p2 * msk, axis=-1, keepdims=True) / 3.0
    v3 = jnp.sum(((p2 - m3) * msk) ** 2, axis=-1, keepdims=True) / 3.0
    pe = ((p2 - m3) * lax.rsqrt(v3 + EPS) * pg[...] + pbe[...]) * msk
    h1 = _mm(x[...], nW1a[...]) + _mm(pe, nW1b[...]) + nb1[...]
    h2 = jnp.maximum(_mm(h1, nW2[...]) + nb2[...], 0.0)
    hh = _ln(h2, ng[...], nbe[...])
    h_out[...] = hh
    a_out[...] = _mm(hh, Wi[...])
    b_out[...] = _mm(hh, Wj[...])


def _edge_encoder_body(eattr, W1, b1, W2, b2, g, be, out):
    a1 = _mm(eattr[...], W1[...]) + b1[...]
    a2 = jnp.maximum(_mmb(a1, W2[...]) + b2[...], 0.0)
    out[...] = _ln(jnp.tanh(a2), g[...], be[...])


def _edge_layer_body(gi, gj, ea, We, b1, W2, b2, g, be, out):
    t = _mmb(ea[...], We[...])
    m1 = jnp.maximum(gi[...] + gj[...] + t + b1[...], 0.0)
    m = _mmb(m1, W2[...]) + b2[...]
    out[...] = _ln(m, g[...], be[...]) + ea[...]


def _node_layer_body(h, p0, p1, W1a, W1b, b1, W2, b2, g, be, out):
    agg = p0[...] + p1[...]
    u1 = jnp.maximum(_mm(h[...], W1a[...]) + _mm(agg, W1b[...]) + b1[...], 0.0)
    u2 = jnp.tanh(_mm(u1, W2[...]) + b2[...])
    out[...] = h[...] + _ln(u2, g[...], be[...])


def _node_proj_body(h, p0, p1, W1a, W1b, b1, W2, b2, g, be, Wi, Wj,
                    out, a_out, b_out):
    agg = p0[...] + p1[...]
    u1 = jnp.maximum(_mm(h[...], W1a[...]) + _mm(agg, W1b[...]) + b1[...], 0.0)
    u2 = jnp.tanh(_mm(u1, W2[...]) + b2[...])
    hn = h[...] + _ln(u2, g[...], be[...])
    out[...] = hn
    a_out[...] = _mm(hn, Wi[...])
    b_out[...] = _mm(hn, Wj[...])


def _dec_body(h, W1, b1, W2, b2, W3, b3, out):
    u1 = jnp.maximum(_mm(h[...], W1[...]) + b1[...], 0.0)
    u2 = _mm(u1, W2[...]) + b2[...]
    out[...] = _mm(u2, W3[...]) + b3[...]


def _row_call(body, n_rows, block, n_in_row, mats, out_shapes):
    grid = n_rows // block

    def rb(arr):
        return pl.BlockSpec((block, arr.shape[-1]), lambda i: (i, 0))

    def whole(arr):
        return pl.BlockSpec(arr.shape, lambda i: tuple(0 for _ in arr.shape))

    def make(*args):
        in_specs = [rb(a) for a in args[:n_in_row]] + [whole(a) for a in mats]
        out_specs = [pl.BlockSpec((block, s[0][-1]), lambda i: (i, 0))
                     for s in out_shapes]
        out_shape = [jax.ShapeDtypeStruct(s, dt) for s, dt in out_shapes]
        single = len(out_shapes) == 1
        return pl.pallas_call(
            body,
            grid=(grid,),
            in_specs=in_specs,
            out_specs=out_specs[0] if single else out_specs,
            out_shape=out_shape[0] if single else out_shape,
        )(*args, *mats)

    return make


def _sc_mesh():
    return plsc.VectorSubcoreMesh(
        core_axis_name="c", subcore_axis_name="s",
        num_cores=SC_CORES, num_subcores=SC_SUBCORES)


def _gather_pair(A, B, dst3, src3, E):
    n_chunks = E // CH
    k_per = (n_chunks + NW - 1) // NW
    H = A.shape[-1]
    dt = A.dtype

    @functools.partial(
        pl.kernel,
        out_type=(jax.ShapeDtypeStruct((E, H), dt),
                  jax.ShapeDtypeStruct((E, H), dt)),
        mesh=_sc_mesh(),
        scratch_types=[
            pltpu.VMEM((k_per, CH), jnp.int32),
            pltpu.VMEM((k_per, CH), jnp.int32),
            pltpu.VMEM((CH, H), dt),
            pltpu.VMEM((CH, H), dt),
            pltpu.VMEM((CH, H), dt),
            pltpu.VMEM((CH, H), dt),
        ] + [pltpu.SemaphoreType.DMA] * 8,
    )
    def k(a_hbm, b_hbm, d_hbm, s_hbm, gi_hbm, gj_hbm,
          dblk, sblk, ra0, ra1, rb0, rb1,
          ga0, ga1, gb0, gb1, wa0, wa1, wb0, wb1):
        wid = lax.axis_index("s") * SC_CORES + lax.axis_index("c")
        pltpu.sync_copy(d_hbm.at[wid], dblk)
        pltpu.sync_copy(s_hbm.at[wid], sblk)
        nv = jnp.maximum(0, jnp.minimum(k_per, n_chunks - wid * k_per))
        ra, rb = (ra0, ra1), (rb0, rb1)
        ga, gb = (ga0, ga1), (gb0, gb1)
        wa, wb = (wa0, wa1), (wb0, wb1)

        def on_parity(j, fn):
            @pl.when(lax.rem(j, 2) == 0)
            def _():
                fn(0)

            @pl.when(lax.rem(j, 2) == 1)
            def _():
                fn(1)

        def issue_gather(j, s):
            pltpu.async_copy(a_hbm.at[dblk.at[j]], ra[s], ga[s])
            pltpu.async_copy(b_hbm.at[sblk.at[j]], rb[s], gb[s])

        def wait_gather(j, s):
            pltpu.make_async_copy(a_hbm.at[dblk.at[j]], ra[s], ga[s]).wait()
            pltpu.make_async_copy(b_hbm.at[sblk.at[j]], rb[s], gb[s]).wait()

        def issue_wb(j, s):
            base = (wid * k_per + j) * CH
            pltpu.async_copy(ra[s], gi_hbm.at[pl.ds(base, CH)], wa[s])
            pltpu.async_copy(rb[s], gj_hbm.at[pl.ds(base, CH)], wb[s])

        def wait_wb(j, s):
            base = (wid * k_per + j) * CH
            pltpu.make_async_copy(ra[s], gi_hbm.at[pl.ds(base, CH)],
                                  wa[s]).wait()
            pltpu.make_async_copy(rb[s], gj_hbm.at[pl.ds(base, CH)],
                                  wb[s]).wait()

        @pl.when(nv > 0)
        def _():
            issue_gather(0, 0)

        def body(j, carry):
            @pl.when(j + 1 < nv)
            def _():
                def ahead(ns):
                    @pl.when(j >= 1)
                    def _():
                        wait_wb(j - 1, ns)

                    issue_gather(j + 1, ns)

                on_parity(j + 1, ahead)

            def cur(s):
                wait_gather(j, s)
                issue_wb(j, s)

            on_parity(j, cur)
            return carry

        lax.fori_loop(0, nv, body, 0)

        @pl.when(nv >= 2)
        def _():
            on_parity(nv - 2, lambda s: wait_wb(nv - 2, s))

        @pl.when(nv >= 1)
        def _():
            on_parity(nv - 1, lambda s: wait_wb(nv - 1, s))

    return k(A, B, dst3, src3)


def _segment_sum_2part(ea, src3, zeros, Np, E):
    n_chunks = E // CH
    k_per = (n_chunks + NW - 1) // NW
    H = ea.shape[-1]
    rpt = Np // SC_SUBCORES

    @functools.partial(
        pl.kernel,
        out_type=jax.ShapeDtypeStruct((2, Np, H), F32),
        mesh=_sc_mesh(),
        scratch_types=[
            pltpu.VMEM((k_per, CH), jnp.int32),
            pltpu.VMEM((CH, H), F32),
            pltpu.VMEM((CH, H), F32),
            pltpu.VMEM_SHARED((Np, H), F32),
        ] + [pltpu.SemaphoreType.DMA] * 4,
    )
    def k(ea_hbm, s_hbm, z_hbm, out_hbm, sblk, rv0, rv1, table,
          gl0, gl1, sc0, sc1):
        cid = lax.axis_index("c")
        sid = lax.axis_index("s")
        wid = sid * SC_CORES + cid
        pltpu.sync_copy(z_hbm.at[pl.ds(sid * rpt, rpt)],
                        table.at[pl.ds(sid * rpt, rpt)])
        pltpu.sync_copy(s_hbm.at[wid], sblk)
        plsc.subcore_barrier()
        nv = jnp.maximum(0, jnp.minimum(k_per, n_chunks - wid * k_per))
        rv = (rv0, rv1)
        gl = (gl0, gl1)
        sc = (sc0, sc1)

        def on_parity(j, fn):
            @pl.when(lax.rem(j, 2) == 0)
            def _():
                fn(0)

            @pl.when(lax.rem(j, 2) == 1)
            def _():
                fn(1)

        def issue_load(j, s):
            base = (wid * k_per + j) * CH
            pltpu.async_copy(ea_hbm.at[pl.ds(base, CH)], rv[s], gl[s])

        def wait_load(j, s):
            base = (wid * k_per + j) * CH
            pltpu.make_async_copy(ea_hbm.at[pl.ds(base, CH)], rv[s],
                                  gl[s]).wait()

        def issue_scat(j, s):
            pltpu.async_copy(rv[s], table.at[sblk.at[j]], sc[s], add=True)

        def wait_scat(j, s):
            pltpu.make_async_copy(rv[s], table.at[sblk.at[j]],
                                  sc[s]).wait()

        @pl.when(nv > 0)
        def _():
            issue_load(0, 0)

        def body(j, carry):
            @pl.when(j + 1 < nv)
            def _():
                def ahead(ns):
                    @pl.when(j >= 1)
                    def _():
                        wait_scat(j - 1, ns)

                    issue_load(j + 1, ns)

                on_parity(j + 1, ahead)

            def cur(s):
                wait_load(j, s)
                issue_scat(j, s)

            on_parity(j, cur)
            return carry

        lax.fori_loop(0, nv, body, 0)

        @pl.when(nv >= 2)
        def _():
            on_parity(nv - 2, lambda s: wait_scat(nv - 2, s))

        @pl.when(nv >= 1)
        def _():
            on_parity(nv - 1, lambda s: wait_scat(nv - 1, s))

        plsc.subcore_barrier()
        pltpu.sync_copy(table.at[pl.ds(sid * rpt, rpt)],
                        out_hbm.at[cid, pl.ds(sid * rpt, rpt)])

    return k(ea, src3, zeros)


def kernel(x, pos, edge_attr, params, edge_index):
    N, DN = x.shape
    E, DE = edge_attr.shape
    DG = pos.shape[-1]
    H = params['node']['W2'].shape[-1]
    HE = params['edge']['W2'].shape[-1]
    OUT = params['dec']['W3'].shape[-1]
    BN = 2000
    BE = 8000

    def row(v):
        return v.reshape(1, -1)

    p = params['pos']
    posp = jnp.pad(pos, ((0, 0), (0, H - DG)))
    pW1 = jnp.pad(p['W1'], ((0, H - DG), (0, 0)))
    pW2 = jnp.pad(p['W2'], ((0, 0), (0, H - DG)))
    pb2 = jnp.pad(row(p['b2']), ((0, 0), (0, H - DG)))
    pg = jnp.pad(row(p['g']), ((0, 0), (0, H - DG)))
    pbe = jnp.pad(row(p['be']), ((0, 0), (0, H - DG)))
    n = params['node']
    nW1a = n['W1'][:DN]
    nW1b = jnp.pad(n['W1'][DN:], ((0, H - DG), (0, 0)))
    lys = params['layers']
    Wis = [lp['eW1'][:H] for lp in lys]
    Wjs = [lp['eW1'][H:2 * H] for lp in lys]
    h, A, B = _row_call(
        _encoder_body, N, BN, 2,
        [pW1, row(p['b1']), pW2, pb2, pg, pbe,
         nW1a, nW1b, row(n['b1']), n['W2'], row(n['b2']),
         row(n['g']), row(n['be']), Wis[0], Wjs[0]],
        [((N, H), F32), ((N, HE), F32), ((N, HE), F32)])(x, posp)

    e = params['edge']
    ea = _row_call(
        _edge_encoder_body, E, BE, 1,
        [e['W1'], row(e['b1']), e['W2'], row(e['b2']), row(e['g']),
         row(e['be'])],
        [((E, HE), F32)])(edge_attr)

    n_chunks = E // CH
    k_per = (n_chunks + NW - 1) // NW
    pad_e = NW * k_per * CH - E
    src = edge_index[0].astype(jnp.int32)
    dst = edge_index[1].astype(jnp.int32)
    dst3 = jnp.pad(dst, (0, pad_e)).reshape(NW, k_per, CH)
    src3 = jnp.pad(src, (0, pad_e)).reshape(NW, k_per, CH)
    Np = 8 * SC_SUBCORES * ((N + 8 * SC_SUBCORES - 1) // (8 * SC_SUBCORES))
    zeros = jnp.zeros((Np, HE), F32)

    for li, lp in enumerate(lys):
        We = lp['eW1'][2 * H:]
        Gi, Gj = _gather_pair(A, B, dst3, src3, E)
        ea = _row_call(
            _edge_layer_body, E, BE, 3,
            [We, row(lp['eb1']), lp['eW2'], row(lp['eb2']), row(lp['eg']),
             row(lp['ebe'])],
            [((E, HE), F32)])(Gi, Gj, ea)
        parts = _segment_sum_2part(ea, src3, zeros, Np, E)
        pv = (parts[0, :N], parts[1, :N])
        if li + 1 < len(lys):
            h, A, B = _row_call(
                _node_proj_body, N, BN, 3,
                [lp['nW1'][:H], lp['nW1'][H:], row(lp['nb1']), lp['nW2'],
                 row(lp['nb2']), row(lp['ng']), row(lp['nbe']),
                 Wis[li + 1], Wjs[li + 1]],
                [((N, H), F32), ((N, HE), F32), ((N, HE), F32)])(h, *pv)
        else:
            h = _row_call(
                _node_layer_body, N, BN, 3,
                [lp['nW1'][:H], lp['nW1'][H:], row(lp['nb1']), lp['nW2'],
                 row(lp['nb2']), row(lp['ng']), row(lp['nbe'])],
                [((N, H), F32)])(h, *pv)

    d = params['dec']
    dW3 = jnp.pad(d['W3'], ((0, 0), (0, H - OUT)))
    db3 = jnp.pad(row(d['b3']), ((0, 0), (0, H - OUT)))
    outp = _row_call(
        _dec_body, N, BN, 1,
        [d['W1'], row(d['b1']), d['W2'], row(d['b2']), dW3, db3],
        [((N, H), F32)])(h)
    return outp[:, :OUT]
